# baseline (speedup 1.0000x reference)
import jax
import jax.numpy as jnp
from jax import lax
from jax.experimental import pallas as pl
from jax.experimental.pallas import tpu as pltpu
from jax.experimental.pallas import tpu_sc as plsc

N_NODES = 10000
N_EDGES = 320000
D = 128

NC = 2
NS = 16
NW = NC * NS

CHUNK = 88
N_CHUNKS = 120
E_PER_TILE = CHUNK * N_CHUNKS
E_PAD = NW * E_PER_TILE

NR = 4
NQ = 8

ACC_ROWS = 10112
ROWS_PER_TILE = ACC_ROWS // NS


def _sc_body(x_hbm, src_hbm, dst_hbm, z_hbm, out_hbm,
             acc, s0, s1, s2, s3, s4, s5, s6, s7,
             d0, d1, d2, d3, d4, d5, d6, d7,
             r0, r1, r2, r3,
             si0, si1, si2, si3, si4, si5, si6, si7,
             sr0, sr1, sr2, sr3, ss0, ss1, ss2, ss3):
  cid = lax.axis_index("c")
  sid = lax.axis_index("s")
  wid = cid * NS + sid
  sidx = (s0, s1, s2, s3, s4, s5, s6, s7)
  didx = (d0, d1, d2, d3, d4, d5, d6, d7)
  rows = (r0, r1, r2, r3)
  isem = (si0, si1, si2, si3, si4, si5, si6, si7)
  rsem = (sr0, sr1, sr2, sr3)
  ssem = (ss0, ss1, ss2, ss3)

  base = wid * E_PER_TILE
  row0 = sid * ROWS_PER_TILE

  def fire_idx(jj, q):
    pltpu.async_copy(src_hbm.at[pl.ds(base + jj * CHUNK, CHUNK)], sidx[q], isem[q])
    pltpu.async_copy(dst_hbm.at[pl.ds(base + jj * CHUNK, CHUNK)], didx[q], isem[q])

  def wait_idx(jj, q):
    pltpu.make_async_copy(src_hbm.at[pl.ds(base + jj * CHUNK, CHUNK)], sidx[q], isem[q]).wait()
    pltpu.make_async_copy(dst_hbm.at[pl.ds(base + jj * CHUNK, CHUNK)], didx[q], isem[q]).wait()

  def fire_gather(p, q):
    pltpu.async_copy(x_hbm.at[sidx[q]], rows[p], rsem[p])

  def wait_gather(p, q):
    pltpu.make_async_copy(x_hbm.at[sidx[q]], rows[p], rsem[p]).wait()

  def fire_scatter(p, q):
    pltpu.async_copy(rows[p], acc.at[didx[q]], ssem[p], add=True)

  def wait_scatter(p, q):
    pltpu.make_async_copy(rows[p], acc.at[didx[q]], ssem[p]).wait()

  @pl.when(cid == 0)
  def _():
    def x_read(o, p, sz):
      pltpu.async_copy(x_hbm.at[pl.ds(row0 + o * CHUNK, sz)],
                       rows[p].at[pl.ds(0, sz)], rsem[p])
    def x_wait(o, p, sz):
      pltpu.make_async_copy(x_hbm.at[pl.ds(row0 + o * CHUNK, sz)],
                            rows[p].at[pl.ds(0, sz)], rsem[p]).wait()
    x_read(0, 0, CHUNK)
    x_read(1, 1, CHUNK)
    for o in range(5):
      p = o % 2
      x_wait(o, p, CHUNK)
      pltpu.sync_copy(rows[p], acc.at[pl.ds(row0 + o * CHUNK, CHUNK)])
      if o < 3:
        x_read(o + 2, p, CHUNK)
    @pl.when(sid < NS - 1)
    def _():
      pltpu.sync_copy(x_hbm.at[pl.ds(row0 + 440, CHUNK)], r0)
      pltpu.sync_copy(r0, acc.at[pl.ds(row0 + 440, CHUNK)])
      pltpu.sync_copy(x_hbm.at[pl.ds(row0 + 528, CHUNK)], r1)
      pltpu.sync_copy(r1, acc.at[pl.ds(row0 + 528, CHUNK)])
      pltpu.sync_copy(x_hbm.at[pl.ds(row0 + 616, 16)], r0.at[pl.ds(0, 16)])
      pltpu.sync_copy(r0.at[pl.ds(0, 16)], acc.at[pl.ds(row0 + 616, 16)])
    @pl.when(sid == NS - 1)
    def _():
      pltpu.sync_copy(x_hbm.at[pl.ds(row0 + 440, 80)], r0.at[pl.ds(0, 80)])
      pltpu.sync_copy(r0.at[pl.ds(0, 80)], acc.at[pl.ds(row0 + 440, 80)])

  @pl.when(cid == 1)
  def _():
    pltpu.sync_copy(z_hbm, r0)
    def init(i, c):
      pltpu.sync_copy(r0, acc.at[pl.ds(row0 + i * CHUNK, CHUNK)])
      return c
    lax.fori_loop(0, 7, init, 0)
    pltpu.sync_copy(r0.at[pl.ds(0, 16)], acc.at[pl.ds(row0 + 616, 16)])

  for q in range(5):
    fire_idx(q, q)
  for q in range(3):
    wait_idx(q, q)
    fire_gather(q, q)

  plsc.subcore_barrier()

  def steps(j, jq, skip_ws=False, skip_fi=False, skip_g=False):
    r, q = jq % NR, jq % NQ
    wait_gather(r, q)
    if not skip_ws:
      wait_scatter((r + 3) % NR, (q + 7) % NQ)
    fire_scatter(r, q)
    if not skip_fi:
      fire_idx(j + 5, (q + 5) % NQ)
    if not skip_g:
      wait_idx(j + 3, (q + 3) % NQ)
      fire_gather((r + 3) % NR, (q + 3) % NQ)

  steps(0, 0, skip_ws=True)
  for j in range(1, NQ):
    steps(j, j)

  def group(g, c):
    j0 = NQ * g
    for b in range(NQ):
      steps(j0 + b, b)
    return c

  lax.fori_loop(1, N_CHUNKS // NQ - 1, group, 0)

  for j in range(N_CHUNKS - NQ, N_CHUNKS):
    steps(j, j, skip_fi=(j + 5 >= N_CHUNKS), skip_g=(j + 3 >= N_CHUNKS))
  wait_scatter((N_CHUNKS - 1) % NR, (N_CHUNKS - 1) % NQ)

  plsc.subcore_barrier()

  obase = cid * ACC_ROWS + row0

  def w_fire(o, p, sz):
    pltpu.async_copy(rows[p].at[pl.ds(0, sz)],
                     out_hbm.at[pl.ds(obase + o * CHUNK, sz)], ssem[p])
  def w_wait(o, p, sz):
    pltpu.make_async_copy(rows[p].at[pl.ds(0, sz)],
                          out_hbm.at[pl.ds(obase + o * CHUNK, sz)], ssem[p]).wait()

  for o in range(7):
    p = o % 2
    if o >= 2:
      w_wait(o - 2, p, CHUNK)
    pltpu.sync_copy(acc.at[pl.ds(row0 + o * CHUNK, CHUNK)], rows[p])
    w_fire(o, p, CHUNK)
  w_wait(5, 1, CHUNK)
  pltpu.sync_copy(acc.at[pl.ds(row0 + 616, 16)], r1.at[pl.ds(0, 16)])
  w_fire(7, 1, 16)
  w_wait(6, 0, CHUNK)
  w_wait(7, 1, 16)


_sc_agg = pl.kernel(
    _sc_body,
    out_type=jax.ShapeDtypeStruct((NC * ACC_ROWS, D), jnp.float32),
    mesh=plsc.VectorSubcoreMesh(
        core_axis_name="c", subcore_axis_name="s",
        num_cores=NC, num_subcores=NS),
    scratch_types=(
        [pltpu.VMEM_SHARED((ACC_ROWS, D), jnp.float32)]
        + [pltpu.VMEM((CHUNK,), jnp.int32)] * NQ
        + [pltpu.VMEM((CHUNK,), jnp.int32)] * NQ
        + [pltpu.VMEM((CHUNK, D), jnp.float32)] * NR
        + [pltpu.SemaphoreType.DMA] * NQ
        + [pltpu.SemaphoreType.DMA] * NR
        + [pltpu.SemaphoreType.DMA] * NR
    ),
)


BR = 1000


def _tc_body(p_ref, w_ref, b_ref, deg_ref, o_ref):
  pool = p_ref[0] + p_ref[1]
  y = jnp.dot(pool, w_ref[...], preferred_element_type=jnp.float32)
  y = (y + b_ref[...]) / deg_ref[...]
  o_ref[...] = jnp.maximum(y, 0.0)


_tc_fin = pl.pallas_call(
    _tc_body,
    grid=(N_NODES // BR,),
    in_specs=[
        pl.BlockSpec((NC, BR, D), lambda i: (0, i, 0)),
        pl.BlockSpec((D, D), lambda i: (0, 0)),
        pl.BlockSpec((1, D), lambda i: (0, 0)),
        pl.BlockSpec((BR, 1), lambda i: (i, 0)),
    ],
    out_specs=pl.BlockSpec((BR, D), lambda i: (i, 0)),
    out_shape=jax.ShapeDtypeStruct((N_NODES, D), jnp.float32),
)


@jax.jit
def kernel(input_tensor, edge_index, node_degree_matrix, weight, bias):
  src = edge_index[0].astype(jnp.int32)
  dst = edge_index[1].astype(jnp.int32)
  npad = E_PAD - N_EDGES
  k = jnp.arange(npad, dtype=jnp.int32)
  src = jnp.concatenate([src, k % N_NODES])
  dst = jnp.concatenate([dst, N_NODES + (k % (ACC_ROWS - N_NODES))])
  zeros = jnp.zeros((CHUNK, D), jnp.float32)
  partials = _sc_agg(input_tensor, src, dst, zeros).reshape(NC, ACC_ROWS, D)
  return _tc_fin(partials, weight, bias.reshape(1, D), node_degree_matrix)


# speedup vs baseline: 1.0165x; 1.0165x over previous
import jax
import jax.numpy as jnp
from jax import lax
from jax.experimental import pallas as pl
from jax.experimental.pallas import tpu as pltpu
from jax.experimental.pallas import tpu_sc as plsc

N_NODES = 10000
N_EDGES = 320000
D = 128

NC = 2
NS = 16
NW = NC * NS

CHUNK = 120
N_CHUNKS = 84
E_PER_TILE = CHUNK * N_CHUNKS
E_PAD = NW * E_PER_TILE

NR = 3
NQ = 6

ACC_ROWS = 10112
ROWS_PER_TILE = ACC_ROWS // NS


def _sc_body(x_hbm, src_hbm, dst_hbm, z_hbm, out_hbm,
             acc, s0, s1, s2, s3, s4, s5, d0, d1, d2, d3, d4, d5,
             r0, r1, r2,
             si0, si1, si2, si3, si4, si5, sr0, sr1, sr2, ss0, ss1, ss2):
  cid = lax.axis_index("c")
  sid = lax.axis_index("s")
  wid = cid * NS + sid
  sidx = (s0, s1, s2, s3, s4, s5)
  didx = (d0, d1, d2, d3, d4, d5)
  rows = (r0, r1, r2)
  isem = (si0, si1, si2, si3, si4, si5)
  rsem = (sr0, sr1, sr2)
  ssem = (ss0, ss1, ss2)

  base = wid * E_PER_TILE
  row0 = sid * ROWS_PER_TILE

  def fire_idx(jj, q):
    pltpu.async_copy(src_hbm.at[pl.ds(base + jj * CHUNK, CHUNK)], sidx[q], isem[q])
    pltpu.async_copy(dst_hbm.at[pl.ds(base + jj * CHUNK, CHUNK)], didx[q], isem[q])

  def wait_idx(jj, q):
    pltpu.make_async_copy(src_hbm.at[pl.ds(base + jj * CHUNK, CHUNK)], sidx[q], isem[q]).wait()
    pltpu.make_async_copy(dst_hbm.at[pl.ds(base + jj * CHUNK, CHUNK)], didx[q], isem[q]).wait()

  def fire_gather(p, q):
    pltpu.async_copy(x_hbm.at[sidx[q]], rows[p], rsem[p])

  def wait_gather(p, q):
    pltpu.make_async_copy(x_hbm.at[sidx[q]], rows[p], rsem[p]).wait()

  def fire_scatter(p, q):
    pltpu.async_copy(rows[p], acc.at[didx[q]], ssem[p], add=True)

  def wait_scatter(p, q):
    pltpu.make_async_copy(rows[p], acc.at[didx[q]], ssem[p]).wait()

  @pl.when(cid == 0)
  def _():
    def x_read(o, p, sz):
      pltpu.async_copy(x_hbm.at[pl.ds(row0 + o * CHUNK, sz)],
                       rows[p].at[pl.ds(0, sz)], rsem[p])
    def x_wait(o, p, sz):
      pltpu.make_async_copy(x_hbm.at[pl.ds(row0 + o * CHUNK, sz)],
                            rows[p].at[pl.ds(0, sz)], rsem[p]).wait()
    x_read(0, 0, CHUNK)
    x_read(1, 1, CHUNK)
    for o in range(4):
      p = o % 2
      x_wait(o, p, CHUNK)
      pltpu.sync_copy(rows[p], acc.at[pl.ds(row0 + o * CHUNK, CHUNK)])
      if o < 2:
        x_read(o + 2, p, CHUNK)
    @pl.when(sid < NS - 1)
    def _():
      pltpu.sync_copy(x_hbm.at[pl.ds(row0 + 480, CHUNK)], r0)
      pltpu.sync_copy(r0, acc.at[pl.ds(row0 + 480, CHUNK)])
      pltpu.sync_copy(x_hbm.at[pl.ds(row0 + 600, 32)], r1.at[pl.ds(0, 32)])
      pltpu.sync_copy(r1.at[pl.ds(0, 32)], acc.at[pl.ds(row0 + 600, 32)])
    @pl.when(sid == NS - 1)
    def _():
      pltpu.sync_copy(x_hbm.at[pl.ds(row0 + 480, 40)], r0.at[pl.ds(0, 40)])
      pltpu.sync_copy(r0.at[pl.ds(0, 40)], acc.at[pl.ds(row0 + 480, 40)])

  @pl.when(cid == 1)
  def _():
    pltpu.sync_copy(z_hbm, r0)
    def init(i, c):
      pltpu.sync_copy(r0, acc.at[pl.ds(row0 + i * CHUNK, CHUNK)])
      return c
    lax.fori_loop(0, 5, init, 0)
    pltpu.sync_copy(r0.at[pl.ds(0, 32)], acc.at[pl.ds(row0 + 600, 32)])

  fire_idx(0, 0)
  fire_idx(1, 1)
  fire_idx(2, 2)
  fire_idx(3, 3)
  wait_idx(0, 0)
  fire_gather(0, 0)
  wait_idx(1, 1)
  fire_gather(1, 1)

  plsc.subcore_barrier()

  def steps(j, jq, skip_ws=False, skip_fi=False, skip_g=False):
    r, q = jq % NR, jq % NQ
    wait_gather(r, q)
    if not skip_ws:
      wait_scatter((r + 2) % NR, (q + 5) % NQ)
    fire_scatter(r, q)
    if not skip_fi:
      fire_idx(j + 4, (q + 4) % NQ)
    if not skip_g:
      wait_idx(j + 2, (q + 2) % NQ)
      fire_gather((r + 2) % NR, (q + 2) % NQ)

  steps(0, 0, skip_ws=True)
  for j in range(1, 6):
    steps(j, j)

  def group(g, c):
    j0 = NQ * g
    for b in range(NQ):
      steps(j0 + b, b)
    return c

  lax.fori_loop(1, N_CHUNKS // NQ - 1, group, 0)

  for j in range(N_CHUNKS - 6, N_CHUNKS):
    steps(j, j, skip_fi=(j + 4 >= N_CHUNKS), skip_g=(j + 2 >= N_CHUNKS))
  wait_scatter((N_CHUNKS - 1) % NR, (N_CHUNKS - 1) % NQ)

  plsc.subcore_barrier()

  obase = cid * ACC_ROWS + row0

  def w_fire(o, p, sz):
    pltpu.async_copy(rows[p].at[pl.ds(0, sz)],
                     out_hbm.at[pl.ds(obase + o * CHUNK, sz)], ssem[p])
  def w_wait(o, p, sz):
    pltpu.make_async_copy(rows[p].at[pl.ds(0, sz)],
                          out_hbm.at[pl.ds(obase + o * CHUNK, sz)], ssem[p]).wait()

  for o in range(5):
    p = o % 2
    if o >= 2:
      w_wait(o - 2, p, CHUNK)
    pltpu.sync_copy(acc.at[pl.ds(row0 + o * CHUNK, CHUNK)], rows[p])
    w_fire(o, p, CHUNK)
  w_wait(3, 1, CHUNK)
  pltpu.sync_copy(acc.at[pl.ds(row0 + 600, 32)], r1.at[pl.ds(0, 32)])
  w_fire(5, 1, 32)
  w_wait(4, 0, CHUNK)
  w_wait(5, 1, 32)


_sc_agg = pl.kernel(
    _sc_body,
    out_type=jax.ShapeDtypeStruct((NC * ACC_ROWS, D), jnp.float32),
    mesh=plsc.VectorSubcoreMesh(
        core_axis_name="c", subcore_axis_name="s",
        num_cores=NC, num_subcores=NS),
    scratch_types=(
        [pltpu.VMEM_SHARED((ACC_ROWS, D), jnp.float32)]
        + [pltpu.VMEM((CHUNK,), jnp.int32)] * NQ
        + [pltpu.VMEM((CHUNK,), jnp.int32)] * NQ
        + [pltpu.VMEM((CHUNK, D), jnp.float32)] * NR
        + [pltpu.SemaphoreType.DMA] * NQ
        + [pltpu.SemaphoreType.DMA] * NR
        + [pltpu.SemaphoreType.DMA] * NR
    ),
)


BR = 1000


def _tc_body(p_ref, w_ref, b_ref, deg_ref, o_ref):
  pool = p_ref[0] + p_ref[1]
  y = jnp.dot(pool, w_ref[...], preferred_element_type=jnp.float32)
  y = (y + b_ref[...]) / deg_ref[...]
  o_ref[...] = jnp.maximum(y, 0.0)


_tc_fin = pl.pallas_call(
    _tc_body,
    grid=(N_NODES // BR,),
    in_specs=[
        pl.BlockSpec((NC, BR, D), lambda i: (0, i, 0)),
        pl.BlockSpec((D, D), lambda i: (0, 0)),
        pl.BlockSpec((1, D), lambda i: (0, 0)),
        pl.BlockSpec((BR, 1), lambda i: (i, 0)),
    ],
    out_specs=pl.BlockSpec((BR, D), lambda i: (i, 0)),
    out_shape=jax.ShapeDtypeStruct((N_NODES, D), jnp.float32),
)


@jax.jit
def kernel(input_tensor, edge_index, node_degree_matrix, weight, bias):
  src = edge_index[0].astype(jnp.int32)
  dst = edge_index[1].astype(jnp.int32)
  npad = E_PAD - N_EDGES
  k = jnp.arange(npad, dtype=jnp.int32)
  src = jnp.concatenate([src, k % N_NODES])
  dst = jnp.concatenate([dst, N_NODES + (k % (ACC_ROWS - N_NODES))])
  zeros = jnp.zeros((CHUNK, D), jnp.float32)
  partials = _sc_agg(input_tensor, src, dst, zeros).reshape(NC, ACC_ROWS, D)
  return _tc_fin(partials, weight, bias.reshape(1, D), node_degree_matrix)


# speedup vs baseline: 1.0207x; 1.0041x over previous
import jax
import jax.numpy as jnp
from jax import lax
from jax.experimental import pallas as pl
from jax.experimental.pallas import tpu as pltpu
from jax.experimental.pallas import tpu_sc as plsc

N_NODES = 10000
N_EDGES = 320000
D = 128

NC = 2
NS = 16
NW = NC * NS

CHUNK = 112
N_CHUNKS = 90
E_PER_TILE = CHUNK * N_CHUNKS
E_PAD = NW * E_PER_TILE

NR = 3
NQ = 6

ACC_ROWS = 10112
ROWS_PER_TILE = ACC_ROWS // NS


def _sc_body(x_hbm, src_hbm, dst_hbm, z_hbm, out_hbm,
             acc, s0, s1, s2, s3, s4, s5, d0, d1, d2, d3, d4, d5,
             r0, r1, r2,
             si0, si1, si2, si3, si4, si5, sr0, sr1, sr2, ss0, ss1, ss2):
  cid = lax.axis_index("c")
  sid = lax.axis_index("s")
  wid = cid * NS + sid
  sidx = (s0, s1, s2, s3, s4, s5)
  didx = (d0, d1, d2, d3, d4, d5)
  rows = (r0, r1, r2)
  isem = (si0, si1, si2, si3, si4, si5)
  rsem = (sr0, sr1, sr2)
  ssem = (ss0, ss1, ss2)

  base = wid * E_PER_TILE
  row0 = sid * ROWS_PER_TILE

  def fire_idx(jj, q):
    pltpu.async_copy(src_hbm.at[pl.ds(base + jj * CHUNK, CHUNK)], sidx[q], isem[q])
    pltpu.async_copy(dst_hbm.at[pl.ds(base + jj * CHUNK, CHUNK)], didx[q], isem[q])

  def wait_idx(jj, q):
    pltpu.make_async_copy(src_hbm.at[pl.ds(base + jj * CHUNK, CHUNK)], sidx[q], isem[q]).wait()
    pltpu.make_async_copy(dst_hbm.at[pl.ds(base + jj * CHUNK, CHUNK)], didx[q], isem[q]).wait()

  def fire_gather(p, q):
    pltpu.async_copy(x_hbm.at[sidx[q]], rows[p], rsem[p])

  def wait_gather(p, q):
    pltpu.make_async_copy(x_hbm.at[sidx[q]], rows[p], rsem[p]).wait()

  def fire_scatter(p, q):
    pltpu.async_copy(rows[p], acc.at[didx[q]], ssem[p], add=True)

  def wait_scatter(p, q):
    pltpu.make_async_copy(rows[p], acc.at[didx[q]], ssem[p]).wait()

  @pl.when(cid == 0)
  def _():
    def x_read(o, p, sz):
      pltpu.async_copy(x_hbm.at[pl.ds(row0 + o * CHUNK, sz)],
                       rows[p].at[pl.ds(0, sz)], rsem[p])
    def x_wait(o, p, sz):
      pltpu.make_async_copy(x_hbm.at[pl.ds(row0 + o * CHUNK, sz)],
                            rows[p].at[pl.ds(0, sz)], rsem[p]).wait()
    x_read(0, 0, CHUNK)
    x_read(1, 1, CHUNK)
    for o in range(4):
      p = o % 2
      x_wait(o, p, CHUNK)
      pltpu.sync_copy(rows[p], acc.at[pl.ds(row0 + o * CHUNK, CHUNK)])
      if o < 2:
        x_read(o + 2, p, CHUNK)
    @pl.when(sid < NS - 1)
    def _():
      pltpu.sync_copy(x_hbm.at[pl.ds(row0 + 448, CHUNK)], r0)
      pltpu.sync_copy(r0, acc.at[pl.ds(row0 + 448, CHUNK)])
      pltpu.sync_copy(x_hbm.at[pl.ds(row0 + 560, 72)], r1.at[pl.ds(0, 72)])
      pltpu.sync_copy(r1.at[pl.ds(0, 72)], acc.at[pl.ds(row0 + 560, 72)])
    @pl.when(sid == NS - 1)
    def _():
      pltpu.sync_copy(x_hbm.at[pl.ds(row0 + 448, 72)], r0.at[pl.ds(0, 72)])
      pltpu.sync_copy(r0.at[pl.ds(0, 72)], acc.at[pl.ds(row0 + 448, 72)])

  @pl.when(cid == 1)
  def _():
    pltpu.sync_copy(z_hbm, r0)
    def init(i, c):
      pltpu.sync_copy(r0, acc.at[pl.ds(row0 + i * CHUNK, CHUNK)])
      return c
    lax.fori_loop(0, 5, init, 0)
    pltpu.sync_copy(r0.at[pl.ds(0, 72)], acc.at[pl.ds(row0 + 560, 72)])

  fire_idx(0, 0)
  fire_idx(1, 1)
  fire_idx(2, 2)
  fire_idx(3, 3)
  wait_idx(0, 0)
  fire_gather(0, 0)
  wait_idx(1, 1)
  fire_gather(1, 1)

  plsc.subcore_barrier()

  def steps(j, jq, skip_ws=False, skip_fi=False, skip_g=False):
    r, q = jq % NR, jq % NQ
    wait_gather(r, q)
    if not skip_ws:
      wait_scatter((r + 2) % NR, (q + 5) % NQ)
    fire_scatter(r, q)
    if not skip_fi:
      fire_idx(j + 4, (q + 4) % NQ)
    if not skip_g:
      wait_idx(j + 2, (q + 2) % NQ)
      fire_gather((r + 2) % NR, (q + 2) % NQ)

  steps(0, 0, skip_ws=True)
  for j in range(1, 6):
    steps(j, j)

  def group(g, c):
    j0 = NQ * g
    for b in range(NQ):
      steps(j0 + b, b)
    return c

  lax.fori_loop(1, N_CHUNKS // NQ - 1, group, 0)

  for j in range(N_CHUNKS - 6, N_CHUNKS):
    steps(j, j, skip_fi=(j + 4 >= N_CHUNKS), skip_g=(j + 2 >= N_CHUNKS))
  wait_scatter((N_CHUNKS - 1) % NR, (N_CHUNKS - 1) % NQ)

  plsc.subcore_barrier()

  obase = cid * ACC_ROWS + row0

  def w_fire(o, p, sz):
    pltpu.async_copy(rows[p].at[pl.ds(0, sz)],
                     out_hbm.at[pl.ds(obase + o * CHUNK, sz)], ssem[p])
  def w_wait(o, p, sz):
    pltpu.make_async_copy(rows[p].at[pl.ds(0, sz)],
                          out_hbm.at[pl.ds(obase + o * CHUNK, sz)], ssem[p]).wait()

  for o in range(5):
    p = o % 2
    if o >= 2:
      w_wait(o - 2, p, CHUNK)
    pltpu.sync_copy(acc.at[pl.ds(row0 + o * CHUNK, CHUNK)], rows[p])
    w_fire(o, p, CHUNK)
  w_wait(3, 1, CHUNK)
  pltpu.sync_copy(acc.at[pl.ds(row0 + 560, 72)], r1.at[pl.ds(0, 72)])
  w_fire(5, 1, 72)
  w_wait(4, 0, CHUNK)
  w_wait(5, 1, 72)


_sc_agg = pl.kernel(
    _sc_body,
    out_type=jax.ShapeDtypeStruct((NC * ACC_ROWS, D), jnp.float32),
    mesh=plsc.VectorSubcoreMesh(
        core_axis_name="c", subcore_axis_name="s",
        num_cores=NC, num_subcores=NS),
    scratch_types=(
        [pltpu.VMEM_SHARED((ACC_ROWS, D), jnp.float32)]
        + [pltpu.VMEM((CHUNK,), jnp.int32)] * NQ
        + [pltpu.VMEM((CHUNK,), jnp.int32)] * NQ
        + [pltpu.VMEM((CHUNK, D), jnp.float32)] * NR
        + [pltpu.SemaphoreType.DMA] * NQ
        + [pltpu.SemaphoreType.DMA] * NR
        + [pltpu.SemaphoreType.DMA] * NR
    ),
)


BR = 1000


def _tc_body(p_ref, w_ref, b_ref, deg_ref, o_ref):
  pool = p_ref[0] + p_ref[1]
  y = jnp.dot(pool, w_ref[...], preferred_element_type=jnp.float32)
  y = (y + b_ref[...]) / deg_ref[...]
  o_ref[...] = jnp.maximum(y, 0.0)


_tc_fin = pl.pallas_call(
    _tc_body,
    grid=(N_NODES // BR,),
    in_specs=[
        pl.BlockSpec((NC, BR, D), lambda i: (0, i, 0)),
        pl.BlockSpec((D, D), lambda i: (0, 0)),
        pl.BlockSpec((1, D), lambda i: (0, 0)),
        pl.BlockSpec((BR, 1), lambda i: (i, 0)),
    ],
    out_specs=pl.BlockSpec((BR, D), lambda i: (i, 0)),
    out_shape=jax.ShapeDtypeStruct((N_NODES, D), jnp.float32),
)


@jax.jit
def kernel(input_tensor, edge_index, node_degree_matrix, weight, bias):
  src = edge_index[0].astype(jnp.int32)
  dst = edge_index[1].astype(jnp.int32)
  npad = E_PAD - N_EDGES
  k = jnp.arange(npad, dtype=jnp.int32)
  src = jnp.concatenate([src, k % N_NODES])
  dst = jnp.concatenate([dst, N_NODES + (k % (ACC_ROWS - N_NODES))])
  zeros = jnp.zeros((CHUNK, D), jnp.float32)
  partials = _sc_agg(input_tensor, src, dst, zeros).reshape(NC, ACC_ROWS, D)
  return _tc_fin(partials, weight, bias.reshape(1, D), node_degree_matrix)
